# baseline (speedup 1.0000x reference)
import functools

import jax
import jax.numpy as jnp
from jax import lax
from jax.experimental import pallas as pl
from jax.experimental.pallas import tpu as pltpu
from jax.experimental.pallas import tpu_sc as plsc

R = 128
C = 32768
NC = 2
NS = 16
NW = NC * NS
RPW = 2
SC_ROWS = NW * RPW
TC_ROWS = R - SC_ROWS
NV = C // 16
NSEG = 16
SEGV = NV // NSEG
SEGW = SEGV * 16
NACC = 4
P1U = 2
VPI1 = NACC * P1U
NIT1 = SEGV // VPI1
P2U = 4
NIT2 = SEGV // P2U
BR = 16

_mesh = plsc.VectorSubcoreMesh(core_axis_name="c", subcore_axis_name="s")


@functools.partial(
    pl.kernel,
    out_type=jax.ShapeDtypeStruct((NW, 16), jnp.int32),
    mesh=_mesh,
    compiler_params=pltpu.CompilerParams(needs_layout_passes=False),
    scratch_types=[
        pltpu.VMEM((C,), jnp.float32),
        pltpu.VMEM((C,), jnp.float32),
        pltpu.VMEM((16,), jnp.int32),
        pltpu.SemaphoreType.DMA,
        pltpu.SemaphoreType.DMA,
    ],
)
def _argmax_sc(x_hbm, out_hbm, buf0, buf1, res_v, sem0, sem1):
    wid = lax.axis_index("s") * NC + lax.axis_index("c")
    lane = lax.iota(jnp.int32, 16)
    bufs = (buf0, buf1)
    sems = (sem0, sem1)
    row0 = wid * RPW

    pltpu.make_async_copy(x_hbm.at[row0], bufs[0], sems[0]).start()

    neg_inf = jnp.full((16,), -jnp.inf, jnp.float32)
    res_vec = jnp.zeros((16,), jnp.int32)
    for rl in range(RPW):
        b = bufs[rl % 2]
        pltpu.make_async_copy(
            x_hbm.at[row0 + rl], b, sems[rl % 2]).wait()
        if rl + 1 < RPW:
            pltpu.make_async_copy(
                x_hbm.at[row0 + rl + 1],
                bufs[(rl + 1) % 2], sems[(rl + 1) % 2]).start()

        def seg_body(s, segmax, b=b):
            sbase = s * SEGW

            def inner(i, accs, b=b, sbase=sbase):
                accs = list(accs)
                for g in range(P1U):
                    for k in range(NACC):
                        off = sbase + (i * VPI1 + g * NACC + k) * 16
                        accs[k] = jnp.maximum(accs[k], b[pl.ds(off, 16)])
                return tuple(accs)

            accs = lax.fori_loop(0, NIT1, inner, (neg_inf,) * NACC)
            mm = jnp.maximum(jnp.maximum(accs[0], accs[1]),
                             jnp.maximum(accs[2], accs[3]))
            ms = jnp.max(mm)
            return jnp.where(lane == s, ms, segmax)

        segmax = lax.fori_loop(0, NSEG, seg_body, neg_inf)

        m = jnp.max(segmax)
        sstar = jnp.min(jnp.where(segmax == m, lane, jnp.int32(NSEG)))
        p2base = sstar * SEGW

        def p2_body(i, cmin, b=b, p2base=p2base, m=m):
            for u in range(P2U):
                off = p2base + (i * P2U + u) * 16
                idx = off + lane
                cmin = jnp.minimum(
                    cmin, jnp.where(b[pl.ds(off, 16)] == m, idx, jnp.int32(0x7FFFFFFF)))
            return cmin

        cmin = lax.fori_loop(0, NIT2, p2_body, jnp.full((16,), 0x7FFFFFFF, jnp.int32))
        best = jnp.min(cmin)
        res_vec = jnp.where(lane == rl, best, res_vec)

    res_v[...] = res_vec
    pltpu.sync_copy(res_v, out_hbm.at[wid])


def _argmax_tc_block(x_ref, o_ref):
    x = x_ref[...]
    m = jnp.max(x, axis=1, keepdims=True)
    ii = lax.broadcasted_iota(jnp.int32, (BR, C), 1)
    cand = jnp.where(x == m, ii, jnp.int32(0x7FFFFFFF))
    o_ref[...] = jnp.min(cand, axis=1, keepdims=True)


_argmax_tc = pl.pallas_call(
    _argmax_tc_block,
    grid=(TC_ROWS // BR,),
    in_specs=[pl.BlockSpec((BR, C), lambda i: (i + SC_ROWS // BR, 0))],
    out_specs=pl.BlockSpec((BR, 1), lambda i: (i + SC_ROWS // BR, 0)),
    out_shape=jax.ShapeDtypeStruct((R, 1), jnp.int32),
)


def kernel(input_data):
    sc_out = _argmax_sc(input_data)
    full = _argmax_tc(input_data)
    sc_part = sc_out[:, :RPW].reshape(SC_ROWS, 1)
    full = lax.dynamic_update_slice(full, sc_part, (0, 0))
    return full.astype(jnp.int64)


# speedup vs baseline: 1.0031x; 1.0031x over previous
import functools

import jax
import jax.numpy as jnp
from jax import lax
from jax.experimental import pallas as pl
from jax.experimental.pallas import tpu as pltpu
from jax.experimental.pallas import tpu_sc as plsc

R = 128
C = 32768
NC = 2
NS = 16
NW = NC * NS
RPW = 2
SC_ROWS = NW * RPW
TC_ROWS = R - SC_ROWS
NV = C // 16
NSEG = 16
SEGV = NV // NSEG
SEGW = SEGV * 16
NACC = 4
P1U = 2
VPI1 = NACC * P1U
NIT1 = SEGV // VPI1
P2U = 4
NIT2 = SEGV // P2U
BR = 16

_mesh = plsc.VectorSubcoreMesh(core_axis_name="c", subcore_axis_name="s")


@functools.partial(
    pl.kernel,
    out_type=jax.ShapeDtypeStruct((NW, 16), jnp.int32),
    mesh=_mesh,
    compiler_params=pltpu.CompilerParams(needs_layout_passes=False),
    scratch_types=[
        pltpu.VMEM((C,), jnp.float32),
        pltpu.VMEM((C,), jnp.float32),
        pltpu.VMEM((16,), jnp.int32),
        pltpu.SemaphoreType.DMA,
        pltpu.SemaphoreType.DMA,
    ],
)
def _argmax_sc(x_hbm, out_hbm, buf0, buf1, res_v, sem0, sem1):
    wid = lax.axis_index("s") * NC + lax.axis_index("c")
    lane = lax.iota(jnp.int32, 16)
    bufs = (buf0, buf1)
    sems = (sem0, sem1)
    row0 = wid * RPW

    pltpu.make_async_copy(x_hbm.at[row0], bufs[0], sems[0]).start()

    neg_inf = jnp.full((16,), -jnp.inf, jnp.float32)
    res_vec = jnp.zeros((16,), jnp.int32)
    for rl in range(RPW):
        b = bufs[rl % 2]
        pltpu.make_async_copy(
            x_hbm.at[row0 + rl], b, sems[rl % 2]).wait()
        if rl + 1 < RPW:
            pltpu.make_async_copy(
                x_hbm.at[row0 + rl + 1],
                bufs[(rl + 1) % 2], sems[(rl + 1) % 2]).start()

        def seg_body(s, segmax, b=b):
            sbase = s * SEGW

            @plsc.parallel_loop(0, SEGV, NACC, unroll=P1U,
                                carry=(neg_inf,) * NACC)
            def accs(v, accs, b=b, sbase=sbase):
                accs = list(accs)
                for k in range(NACC):
                    off = sbase + (v + k) * 16
                    accs[k] = jnp.maximum(accs[k], b[pl.ds(off, 16)])
                return tuple(accs)
            mm = jnp.maximum(jnp.maximum(accs[0], accs[1]),
                             jnp.maximum(accs[2], accs[3]))
            ms = jnp.max(mm)
            return jnp.where(lane == s, ms, segmax)

        segmax = lax.fori_loop(0, NSEG, seg_body, neg_inf)

        m = jnp.max(segmax)
        sstar = jnp.min(jnp.where(segmax == m, lane, jnp.int32(NSEG)))
        p2base = sstar * SEGW

        @plsc.parallel_loop(0, SEGV, P2U, unroll=2,
                            carry=jnp.full((16,), 0x7FFFFFFF, jnp.int32))
        def cmin(v, cmin, b=b, p2base=p2base, m=m):
            for u in range(P2U):
                off = p2base + (v + u) * 16
                idx = off + lane
                cmin = jnp.minimum(
                    cmin, jnp.where(b[pl.ds(off, 16)] == m, idx,
                                    jnp.int32(0x7FFFFFFF)))
            return cmin
        best = jnp.min(cmin)
        res_vec = jnp.where(lane == rl, best, res_vec)

    res_v[...] = res_vec
    pltpu.sync_copy(res_v, out_hbm.at[wid])


def _argmax_tc_block(x_ref, o_ref):
    x = x_ref[...]
    m = jnp.max(x, axis=1, keepdims=True)
    ii = lax.broadcasted_iota(jnp.int32, (BR, C), 1)
    cand = jnp.where(x == m, ii, jnp.int32(0x7FFFFFFF))
    o_ref[...] = jnp.min(cand, axis=1, keepdims=True)


_argmax_tc = pl.pallas_call(
    _argmax_tc_block,
    grid=(TC_ROWS // BR,),
    in_specs=[pl.BlockSpec((BR, C), lambda i: (i + SC_ROWS // BR, 0))],
    out_specs=pl.BlockSpec((BR, 1), lambda i: (i + SC_ROWS // BR, 0)),
    out_shape=jax.ShapeDtypeStruct((R, 1), jnp.int32),
)


def kernel(input_data):
    sc_out = _argmax_sc(input_data)
    full = _argmax_tc(input_data)
    sc_part = sc_out[:, :RPW].reshape(SC_ROWS, 1)
    full = lax.dynamic_update_slice(full, sc_part, (0, 0))
    return full.astype(jnp.int64)


# speedup vs baseline: 1.0188x; 1.0157x over previous
import functools

import jax
import jax.numpy as jnp
from jax import lax
from jax.experimental import pallas as pl
from jax.experimental.pallas import tpu as pltpu
from jax.experimental.pallas import tpu_sc as plsc

R = 128
C = 32768
NC = 2
NS = 16
NW = NC * NS
RPW = 2
SC_ROWS = NW * RPW
TC_ROWS = R - SC_ROWS
NACC = 4
NGRP = 2
VPI = NACC * NGRP
NIT = (C // 16) // VPI
BR = 16

_mesh = plsc.VectorSubcoreMesh(core_axis_name="c", subcore_axis_name="s")


@functools.partial(
    pl.kernel,
    out_type=jax.ShapeDtypeStruct((NW, 16), jnp.int32),
    mesh=_mesh,
    compiler_params=pltpu.CompilerParams(needs_layout_passes=False),
    scratch_types=[
        pltpu.VMEM((C,), jnp.float32),
        pltpu.VMEM((C,), jnp.float32),
        pltpu.VMEM((16,), jnp.int32),
        pltpu.SemaphoreType.DMA,
        pltpu.SemaphoreType.DMA,
    ],
)
def _argmax_sc(x_hbm, out_hbm, buf0, buf1, res_v, sem0, sem1):
    wid = lax.axis_index("s") * NC + lax.axis_index("c")
    lane = lax.iota(jnp.int32, 16)
    bufs = (buf0, buf1)
    sems = (sem0, sem1)
    row0 = wid * RPW

    pltpu.make_async_copy(x_hbm.at[row0], bufs[0], sems[0]).start()

    res_vec = jnp.zeros((16,), jnp.int32)
    for rl in range(RPW):
        b = bufs[rl % 2]
        pltpu.make_async_copy(
            x_hbm.at[row0 + rl], b, sems[rl % 2]).wait()
        if rl + 1 < RPW:
            pltpu.make_async_copy(
                x_hbm.at[row0 + rl + 1],
                bufs[(rl + 1) % 2], sems[(rl + 1) % 2]).start()

        neg_inf = jnp.full((16,), -jnp.inf, jnp.float32)
        zero = jnp.zeros((16,), jnp.int32)
        init = (neg_inf,) * NACC + (zero,) * NACC

        def body(i, carry, b=b):
            cmax = list(carry[:NACC])
            crec = list(carry[NACC:])
            base = i * VPI
            for g in range(NGRP):
                for k in range(NACC):
                    gi = base + g * NACC + k
                    val = b[pl.ds(gi * 16, 16)]
                    m = val > cmax[k]
                    cmax[k] = jnp.where(m, val, cmax[k])
                    crec[k] = jnp.where(m, gi, crec[k])
            return tuple(cmax) + tuple(crec)

        acc = lax.fori_loop(0, NIT, body, init)
        cmax = list(acc[:NACC])
        crec = list(acc[NACC:])

        n = NACC
        while n > 1:
            for k in range(n // 2):
                av, bv = cmax[2 * k], cmax[2 * k + 1]
                ar, br = crec[2 * k], crec[2 * k + 1]
                take_a = (av > bv) | ((av == bv) & (ar < br))
                cmax[k] = jnp.where(take_a, av, bv)
                crec[k] = jnp.where(take_a, ar, br)
            n //= 2

        m = jnp.max(cmax[0])
        idx = crec[0] * 16 + lane
        cand = jnp.where(cmax[0] == m, idx, jnp.int32(0x7FFFFFFF))
        best = jnp.min(cand)
        res_vec = jnp.where(lane == rl, best, res_vec)

    res_v[...] = res_vec
    pltpu.sync_copy(res_v, out_hbm.at[wid])


def _argmax_tc_block(x_ref, o_ref):
    x = x_ref[...]
    m = jnp.max(x, axis=1, keepdims=True)
    ii = lax.broadcasted_iota(jnp.int32, (BR, C), 1)
    cand = jnp.where(x == m, ii, jnp.int32(0x7FFFFFFF))
    o_ref[...] = jnp.min(cand, axis=1, keepdims=True)


_argmax_tc = pl.pallas_call(
    _argmax_tc_block,
    grid=(TC_ROWS // BR,),
    in_specs=[pl.BlockSpec((BR, C), lambda i: (i + SC_ROWS // BR, 0))],
    out_specs=pl.BlockSpec((BR, 1), lambda i: (i + SC_ROWS // BR, 0)),
    out_shape=jax.ShapeDtypeStruct((R, 1), jnp.int32),
)


def kernel(input_data):
    sc_out = _argmax_sc(input_data)
    full = _argmax_tc(input_data)
    sc_part = sc_out[:, :RPW].reshape(SC_ROWS, 1)
    full = lax.dynamic_update_slice(full, sc_part, (0, 0))
    return full.astype(jnp.int64)
